# Initial kernel scaffold; baseline (speedup 1.0000x reference)
import jax
import jax.numpy as jnp
from jax.experimental import pallas as pl


def kernel(logits_src, logits_tgt, edge_index_src, edge_index_tgt, cluster_labels_src, cluster_labels_tgt, depth_src, depth_tgt, raw):
    raise NotImplementedError("write your pallas kernel here")


# speedup vs baseline: 23.2922x; 23.2922x over previous
import functools

import jax
import jax.numpy as jnp
from jax import lax
from jax.experimental import pallas as pl
from jax.experimental.pallas import tpu as pltpu
from jax.experimental.pallas import tpu_sc as plsc

N = 50000
C = 32
E = 800000

NW = 32
EPW = E // NW
CHUNK = 1000
NCHUNK = EPW // CHUNK

K_RANK = 1440000


def _prep_body(ls_ref, lt_ref, cs_ref, ct_ref, ds_ref, dt_ref,
               ps_ref, pt_ref, ms_ref, mt_ref):
    for l_ref, p_ref in ((ls_ref, ps_ref), (lt_ref, pt_ref)):
        x = l_ref[...]
        m = jnp.max(x, axis=1, keepdims=True)
        e = jnp.exp(x - m)
        p_ref[...] = e / jnp.sum(e, axis=1, keepdims=True)
    @pl.when(pl.program_id(0) == 0)
    def _():
        ms_ref[...] = cs_ref[...] * 16 + ds_ref[...].astype(jnp.int32)
        mt_ref[...] = ct_ref[...] * 16 + dt_ref[...].astype(jnp.int32)


_PBS = 2000


def _prep(logits_src, logits_tgt, cl_src, cl_tgt, d_src, d_tgt):
    r16 = lambda a: a.reshape(N // 16, 16)
    lspec = pl.BlockSpec((_PBS, C), lambda i: (i, 0))
    mspec = pl.BlockSpec((N // 16, 16), lambda i: (0, 0))
    outs = pl.pallas_call(
        _prep_body,
        grid=(N // _PBS,),
        in_specs=[lspec, lspec, mspec, mspec, mspec, mspec],
        out_specs=(lspec, lspec, mspec, mspec),
        out_shape=(
            jax.ShapeDtypeStruct((N, C), jnp.float32),
            jax.ShapeDtypeStruct((N, C), jnp.float32),
            jax.ShapeDtypeStruct((N // 16, 16), jnp.int32),
            jax.ShapeDtypeStruct((N // 16, 16), jnp.int32),
        ),
    )(logits_src, logits_tgt, r16(cl_src), r16(cl_tgt), r16(d_src), r16(d_tgt))
    ps, pt, ms, mt = outs
    return ps, pt, ms.reshape(N), mt.reshape(N)


def _edges_body(ps_hbm, pt_hbm, ms_hbm, mt_hbm, eis_hbm, eit_hbm,
                sq_hbm, fl_hbm,
                idx_i, idx_j, rows_i, rows_j, mi_v, mj_v, sq_v, fl_v, sem):
    wid = lax.axis_index("s") * 2 + lax.axis_index("c")
    base_w = wid * EPW

    def do_domain(probs_hbm, meta_hbm, ei_hbm, out_base):
        def chunk_body(cidx, carry):
            base = base_w + cidx * CHUNK
            pltpu.sync_copy(ei_hbm.at[pl.ds(base, CHUNK)], idx_i)
            pltpu.sync_copy(ei_hbm.at[pl.ds(E + base, CHUNK)], idx_j)
            cp1 = pltpu.async_copy(probs_hbm.at[idx_i], rows_i, sem)
            cp2 = pltpu.async_copy(probs_hbm.at[idx_j], rows_j, sem)
            cp3 = pltpu.async_copy(meta_hbm.at[idx_i], mi_v, sem)
            cp4 = pltpu.async_copy(meta_hbm.at[idx_j], mj_v, sem)
            cp1.wait()
            cp2.wait()
            cp3.wait()
            cp4.wait()

            lane = lax.iota(jnp.int32, 16)

            def group_body(g, c):
                off = jnp.minimum(g * 16, CHUNK - 16)
                eidx = off + lane
                acc = jnp.zeros((16,), jnp.float32)
                for comp in range(C):
                    cvec = jnp.full((16,), comp, jnp.int32)
                    pi = plsc.load_gather(rows_i, [eidx, cvec])
                    pj = plsc.load_gather(rows_j, [eidx, cvec])
                    d = pi - pj
                    acc = d * d + acc
                sq_v[pl.ds(off, 16)] = acc

                mi = mi_v[pl.ds(off, 16)]
                mj = mj_v[pl.ds(off, 16)]
                same = (lax.shift_right_logical(mi, 4)
                        == lax.shift_right_logical(mj, 4)).astype(jnp.int32)
                dd = jnp.abs((mi & 15) - (mj & 15))
                near = (dd < 3).astype(jnp.int32)
                fl_v[pl.ds(off, 16)] = same + 2 * near
                return c

            lax.fori_loop(0, (CHUNK + 15) // 16, group_body, 0)

            pltpu.sync_copy(sq_v, sq_hbm.at[pl.ds(out_base + base, CHUNK)])
            pltpu.sync_copy(fl_v, fl_hbm.at[pl.ds(out_base + base, CHUNK)])
            return carry

        lax.fori_loop(0, NCHUNK, chunk_body, 0)

    do_domain(ps_hbm, ms_hbm, eis_hbm, 0)
    do_domain(pt_hbm, mt_hbm, eit_hbm, E)


def _edges(probs_src, probs_tgt, meta_src, meta_tgt, ei_src, ei_tgt):
    mesh = plsc.VectorSubcoreMesh(core_axis_name="c", subcore_axis_name="s")
    fn = functools.partial(
        pl.kernel,
        out_type=(
            jax.ShapeDtypeStruct((2 * E,), jnp.float32),
            jax.ShapeDtypeStruct((2 * E,), jnp.int32),
        ),
        mesh=mesh,
        scratch_types=[
            pltpu.VMEM((CHUNK,), jnp.int32),
            pltpu.VMEM((CHUNK,), jnp.int32),
            pltpu.VMEM((CHUNK, C), jnp.float32),
            pltpu.VMEM((CHUNK, C), jnp.float32),
            pltpu.VMEM((CHUNK,), jnp.int32),
            pltpu.VMEM((CHUNK,), jnp.int32),
            pltpu.VMEM((CHUNK,), jnp.float32),
            pltpu.VMEM((CHUNK,), jnp.int32),
            pltpu.SemaphoreType.DMA,
        ],
        compiler_params=pltpu.CompilerParams(
            needs_layout_passes=False, use_tc_tiling_on_sc=False),
    )(_edges_body)
    return fn(probs_src, probs_tgt, meta_src, meta_tgt,
              ei_src.reshape(2 * E), ei_tgt.reshape(2 * E))


_FCH = 25


def _fin_body(sq_ref, fl_ref, raw_ref, out_ref):
    nch = sq_ref.shape[0]

    def acc_body(i, c):
        sum_c, cnt_c, sum_d, cnt_d = c
        s = sq_ref[i]
        fl = fl_ref[i]
        pd = jnp.sqrt(s)
        cs = (fl & 1) > 0
        dn = (fl & 2) > 0
        sum_c += jnp.sum(jnp.where(cs, pd, 0.0))
        cnt_c += jnp.sum(cs.astype(jnp.float32))
        sum_d += jnp.sum(jnp.where(dn, pd, 0.0))
        cnt_d += jnp.sum(dn.astype(jnp.float32))
        return sum_c, cnt_c, sum_d, cnt_d

    z = jnp.float32(0.0)
    sum_c, cnt_c, sum_d, cnt_d = lax.fori_loop(
        0, nch, acc_body, (z, z, z, z))

    def bs_body(_, lohi):
        lo, hi = lohi
        mid = lo + lax.shift_right_logical(hi - lo, 1)

        def cnt_body(i, c):
            bits = lax.bitcast_convert_type(sq_ref[i], jnp.int32)
            return c + jnp.sum((bits <= mid).astype(jnp.int32))

        cnt = lax.fori_loop(0, nch, cnt_body, jnp.int32(0))
        ge = cnt >= K_RANK
        return jnp.where(ge, lo, mid), jnp.where(ge, mid, hi)

    lo0 = jnp.int32(-1)
    hi0 = jnp.int32(0x40800000)
    _, b_hi = lax.fori_loop(0, 31, bs_body, (lo0, hi0))

    def jmp_body(i, c):
        sum_j, cnt_j = c
        s = sq_ref[i]
        bits = lax.bitcast_convert_type(s, jnp.int32)
        jm = bits > b_hi
        sum_j += jnp.sum(jnp.where(jm, jnp.sqrt(s), 0.0))
        cnt_j += jnp.sum(jm.astype(jnp.float32))
        return sum_j, cnt_j

    sum_j, cnt_j = lax.fori_loop(0, nch, jmp_body, (z, z))

    def mmean(s, c):
        return jnp.where(c > 0, s / jnp.maximum(c, 1.0), 0.0)

    p_c = mmean(sum_c, cnt_c)
    p_d = mmean(sum_d, cnt_d)
    p_j = mmean(sum_j, cnt_j)

    r0 = raw_ref[0]
    r1 = raw_ref[1]
    r2 = raw_ref[2]
    rm = jnp.maximum(r0, jnp.maximum(r1, r2))
    e0 = jnp.exp(r0 - rm)
    e1 = jnp.exp(r1 - rm)
    e2 = jnp.exp(r2 - rm)
    es = e0 + e1 + e2
    total = (e0 * p_c + e1 * p_d + e2 * p_j) / es
    out_ref[...] = jnp.broadcast_to(total, (1, 1))


def _finalize(sq, fl, raw):
    out = pl.pallas_call(
        _fin_body,
        out_shape=jax.ShapeDtypeStruct((1, 1), jnp.float32),
        in_specs=[
            pl.BlockSpec(memory_space=pltpu.VMEM),
            pl.BlockSpec(memory_space=pltpu.VMEM),
            pl.BlockSpec(memory_space=pltpu.SMEM),
        ],
        out_specs=pl.BlockSpec(memory_space=pltpu.VMEM),
    )(sq.reshape(_FCH, 2 * E // _FCH // 128, 128),
      fl.reshape(_FCH, 2 * E // _FCH // 128, 128), raw)
    return out[0, 0]


def kernel(logits_src, logits_tgt, edge_index_src, edge_index_tgt,
           cluster_labels_src, cluster_labels_tgt, depth_src, depth_tgt, raw):
    probs_src, probs_tgt, meta_src, meta_tgt = _prep(
        logits_src, logits_tgt, cluster_labels_src, cluster_labels_tgt,
        depth_src, depth_tgt)
    sq, fl = _edges(probs_src, probs_tgt, meta_src, meta_tgt,
                    edge_index_src, edge_index_tgt)
    return _finalize(sq, fl, raw)


# speedup vs baseline: 68.9785x; 2.9614x over previous
import functools

import jax
import jax.numpy as jnp
from jax import lax
from jax.experimental import pallas as pl
from jax.experimental.pallas import tpu as pltpu
from jax.experimental.pallas import tpu_sc as plsc

N = 50000
C = 32
E = 800000

NW = 32
EPW = E // NW
CHUNK = 1000
NCHUNK = EPW // CHUNK

K_RANK = 1440000


def _prep_body(ls_ref, lt_ref, cs_ref, ct_ref, ds_ref, dt_ref,
               ps_ref, pt_ref, ms_ref, mt_ref):
    for l_ref, p_ref in ((ls_ref, ps_ref), (lt_ref, pt_ref)):
        x = l_ref[...]
        m = jnp.max(x, axis=1, keepdims=True)
        e = jnp.exp(x - m)
        p = e / jnp.sum(e, axis=1, keepdims=True)
        u = lax.bitcast_convert_type(p, jnp.int32)
        r = lax.shift_right_logical(
            u + 0x7FFF + (lax.shift_right_logical(u, 16) & 1), 16)
        lo = r[:, 0:16]
        hi = r[:, 16:32]
        p_ref[...] = lo | lax.shift_left(hi, 16)
    @pl.when(pl.program_id(0) == 0)
    def _():
        ms_ref[...] = cs_ref[...] * 16 + ds_ref[...].astype(jnp.int32)
        mt_ref[...] = ct_ref[...] * 16 + dt_ref[...].astype(jnp.int32)


_PBS = 2000


def _prep(logits_src, logits_tgt, cl_src, cl_tgt, d_src, d_tgt):
    r16 = lambda a: a.reshape(N // 16, 16)
    lspec = pl.BlockSpec((_PBS, C), lambda i: (i, 0))
    pspec = pl.BlockSpec((_PBS, C // 2), lambda i: (i, 0))
    mspec = pl.BlockSpec((N // 16, 16), lambda i: (0, 0))
    outs = pl.pallas_call(
        _prep_body,
        grid=(N // _PBS,),
        in_specs=[lspec, lspec, mspec, mspec, mspec, mspec],
        out_specs=(pspec, pspec, mspec, mspec),
        out_shape=(
            jax.ShapeDtypeStruct((N, C // 2), jnp.int32),
            jax.ShapeDtypeStruct((N, C // 2), jnp.int32),
            jax.ShapeDtypeStruct((N // 16, 16), jnp.int32),
            jax.ShapeDtypeStruct((N // 16, 16), jnp.int32),
        ),
    )(logits_src, logits_tgt, r16(cl_src), r16(cl_tgt), r16(d_src), r16(d_tgt))
    ps, pt, ms, mt = outs
    return ps, pt, ms.reshape(N), mt.reshape(N)


def _edges_body(ps_hbm, pt_hbm, ms_hbm, mt_hbm, eis_hbm, eit_hbm,
                sq_hbm, fl_hbm,
                idx_i, idx_j, rows_i, rows_j, mi_v, mj_v, sq_v, fl_v, sems):
    wid = lax.axis_index("s") * 2 + lax.axis_index("c")
    base_w = wid * EPW
    lane = lax.iota(jnp.int32, 16)
    himask = jnp.int32(-65536)

    def do_domain(probs_hbm, meta_hbm, ei_hbm, out_base):
        def issue(cidx, b):
            base = base_w + cidx * CHUNK
            pltpu.sync_copy(ei_hbm.at[pl.ds(base, CHUNK)], idx_i[b])
            pltpu.sync_copy(ei_hbm.at[pl.ds(E + base, CHUNK)], idx_j[b])
            pltpu.async_copy(probs_hbm.at[idx_i[b]], rows_i[b], sems[b])
            pltpu.async_copy(probs_hbm.at[idx_j[b]], rows_j[b], sems[b])
            pltpu.async_copy(meta_hbm.at[idx_i[b]], mi_v[b], sems[b])
            pltpu.async_copy(meta_hbm.at[idx_j[b]], mj_v[b], sems[b])

        def consume(cidx, b):
            pltpu.make_async_copy(probs_hbm.at[idx_i[b]], rows_i[b],
                                  sems[b]).wait()
            pltpu.make_async_copy(probs_hbm.at[idx_j[b]], rows_j[b],
                                  sems[b]).wait()
            pltpu.make_async_copy(meta_hbm.at[idx_i[b]], mi_v[b],
                                  sems[b]).wait()
            pltpu.make_async_copy(meta_hbm.at[idx_j[b]], mj_v[b],
                                  sems[b]).wait()
            ri = rows_i[b]
            rj = rows_j[b]

            def group_body(g, c):
                off = jnp.minimum(g * 16, CHUNK - 16)
                eidx = off + lane
                acc0 = jnp.zeros((16,), jnp.float32)
                acc1 = jnp.zeros((16,), jnp.float32)
                for comp in range(C // 2):
                    cvec = jnp.full((16,), comp, jnp.int32)
                    gi = plsc.load_gather(ri, [eidx, cvec])
                    gj = plsc.load_gather(rj, [eidx, cvec])
                    li = plsc.bitcast(lax.shift_left(gi, 16), jnp.float32)
                    lj = plsc.bitcast(lax.shift_left(gj, 16), jnp.float32)
                    hi = plsc.bitcast(gi & himask, jnp.float32)
                    hj = plsc.bitcast(gj & himask, jnp.float32)
                    d0 = li - lj
                    d1 = hi - hj
                    acc0 = d0 * d0 + acc0
                    acc1 = d1 * d1 + acc1
                sq_v[pl.ds(off, 16)] = acc0 + acc1

                mi = mi_v[b][pl.ds(off, 16)]
                mj = mj_v[b][pl.ds(off, 16)]
                same = (lax.shift_right_logical(mi, 4)
                        == lax.shift_right_logical(mj, 4)).astype(jnp.int32)
                dd = jnp.abs((mi & 15) - (mj & 15))
                near = (dd < 3).astype(jnp.int32)
                fl_v[pl.ds(off, 16)] = same + 2 * near
                return c

            lax.fori_loop(0, (CHUNK + 15) // 16, group_body, 0)

            base = base_w + cidx * CHUNK
            pltpu.sync_copy(sq_v, sq_hbm.at[pl.ds(out_base + base, CHUNK)])
            pltpu.sync_copy(fl_v, fl_hbm.at[pl.ds(out_base + base, CHUNK)])

        issue(0, 0)

        def pair_body(t, carry):
            c0 = t * 2
            issue(c0 + 1, 1)
            consume(c0, 0)
            issue(c0 + 2, 0)
            consume(c0 + 1, 1)
            return carry

        lax.fori_loop(0, (NCHUNK - 1) // 2, pair_body, 0)
        consume(NCHUNK - 1, 0)

    do_domain(ps_hbm, ms_hbm, eis_hbm, 0)
    do_domain(pt_hbm, mt_hbm, eit_hbm, E)


def _edges(probs_src, probs_tgt, meta_src, meta_tgt, ei_src, ei_tgt):
    mesh = plsc.VectorSubcoreMesh(core_axis_name="c", subcore_axis_name="s")
    buf2 = lambda shape, dt: [pltpu.VMEM(shape, dt), pltpu.VMEM(shape, dt)]
    fn = functools.partial(
        pl.kernel,
        out_type=(
            jax.ShapeDtypeStruct((2 * E,), jnp.float32),
            jax.ShapeDtypeStruct((2 * E,), jnp.int32),
        ),
        mesh=mesh,
        scratch_types=[
            buf2((CHUNK,), jnp.int32),
            buf2((CHUNK,), jnp.int32),
            buf2((CHUNK, C // 2), jnp.int32),
            buf2((CHUNK, C // 2), jnp.int32),
            buf2((CHUNK,), jnp.int32),
            buf2((CHUNK,), jnp.int32),
            pltpu.VMEM((CHUNK,), jnp.float32),
            pltpu.VMEM((CHUNK,), jnp.int32),
            [pltpu.SemaphoreType.DMA, pltpu.SemaphoreType.DMA],
        ],
        compiler_params=pltpu.CompilerParams(
            needs_layout_passes=False, use_tc_tiling_on_sc=False),
    )(_edges_body)
    return fn(probs_src, probs_tgt, meta_src, meta_tgt,
              ei_src.reshape(2 * E), ei_tgt.reshape(2 * E))


_FCH = 25


def _fin_body(sq_ref, fl_ref, raw_ref, out_ref):
    nch = sq_ref.shape[0]

    def acc_body(i, c):
        sum_c, cnt_c, sum_d, cnt_d = c
        s = sq_ref[i]
        fl = fl_ref[i]
        pd = jnp.sqrt(s)
        cs = (fl & 1) > 0
        dn = (fl & 2) > 0
        sum_c += jnp.sum(jnp.where(cs, pd, 0.0))
        cnt_c += jnp.sum(cs.astype(jnp.float32))
        sum_d += jnp.sum(jnp.where(dn, pd, 0.0))
        cnt_d += jnp.sum(dn.astype(jnp.float32))
        return sum_c, cnt_c, sum_d, cnt_d

    z = jnp.float32(0.0)
    sum_c, cnt_c, sum_d, cnt_d = lax.fori_loop(
        0, nch, acc_body, (z, z, z, z))

    def bs_body(_, lohi):
        lo, hi = lohi
        mid = lo + lax.shift_right_logical(hi - lo, 1)

        def cnt_body(i, c):
            bits = lax.bitcast_convert_type(sq_ref[i], jnp.int32)
            return c + jnp.sum((bits <= mid).astype(jnp.int32))

        cnt = lax.fori_loop(0, nch, cnt_body, jnp.int32(0))
        ge = cnt >= K_RANK
        return jnp.where(ge, lo, mid), jnp.where(ge, mid, hi)

    lo0 = jnp.int32(-1)
    hi0 = jnp.int32(0x40800000)
    _, b_hi = lax.fori_loop(0, 31, bs_body, (lo0, hi0))

    def jmp_body(i, c):
        sum_j, cnt_j = c
        s = sq_ref[i]
        bits = lax.bitcast_convert_type(s, jnp.int32)
        jm = bits > b_hi
        sum_j += jnp.sum(jnp.where(jm, jnp.sqrt(s), 0.0))
        cnt_j += jnp.sum(jm.astype(jnp.float32))
        return sum_j, cnt_j

    sum_j, cnt_j = lax.fori_loop(0, nch, jmp_body, (z, z))

    def mmean(s, c):
        return jnp.where(c > 0, s / jnp.maximum(c, 1.0), 0.0)

    p_c = mmean(sum_c, cnt_c)
    p_d = mmean(sum_d, cnt_d)
    p_j = mmean(sum_j, cnt_j)

    r0 = raw_ref[0]
    r1 = raw_ref[1]
    r2 = raw_ref[2]
    rm = jnp.maximum(r0, jnp.maximum(r1, r2))
    e0 = jnp.exp(r0 - rm)
    e1 = jnp.exp(r1 - rm)
    e2 = jnp.exp(r2 - rm)
    es = e0 + e1 + e2
    total = (e0 * p_c + e1 * p_d + e2 * p_j) / es
    out_ref[...] = jnp.broadcast_to(total, (1, 1))


def _finalize(sq, fl, raw):
    out = pl.pallas_call(
        _fin_body,
        out_shape=jax.ShapeDtypeStruct((1, 1), jnp.float32),
        in_specs=[
            pl.BlockSpec(memory_space=pltpu.VMEM),
            pl.BlockSpec(memory_space=pltpu.VMEM),
            pl.BlockSpec(memory_space=pltpu.SMEM),
        ],
        out_specs=pl.BlockSpec(memory_space=pltpu.VMEM),
    )(sq.reshape(_FCH, 2 * E // _FCH // 128, 128),
      fl.reshape(_FCH, 2 * E // _FCH // 128, 128), raw)
    return out[0, 0]


def kernel(logits_src, logits_tgt, edge_index_src, edge_index_tgt,
           cluster_labels_src, cluster_labels_tgt, depth_src, depth_tgt, raw):
    probs_src, probs_tgt, meta_src, meta_tgt = _prep(
        logits_src, logits_tgt, cluster_labels_src, cluster_labels_tgt,
        depth_src, depth_tgt)
    sq, fl = _edges(probs_src, probs_tgt, meta_src, meta_tgt,
                    edge_index_src, edge_index_tgt)
    return _finalize(sq, fl, raw)
